# baseline (speedup 1.0000x reference)
import jax
import jax.numpy as jnp
from jax.experimental import pallas as pl
from jax.experimental.pallas import tpu as pltpu

T, D, F, E, K = 32, 1024, 704, 64, 8
NBUF = 4


def _moe_kernel(se_ref, rw_ref, s0_ref, s1_ref, s2_ref, x_ref,
                w0_hbm, w1_hbm, w2_hbm, o_ref,
                w0_buf, w1_buf, w2_buf, sems):
    e = pl.program_id(0)
    slot = jax.lax.rem(e, NBUF)

    def issue(step, slot):
        pltpu.make_async_copy(w0_hbm.at[step], w0_buf.at[slot], sems.at[0, slot]).start()
        pltpu.make_async_copy(w1_hbm.at[step], w1_buf.at[slot], sems.at[1, slot]).start()
        pltpu.make_async_copy(w2_hbm.at[step], w2_buf.at[slot], sems.at[2, slot]).start()

    @pl.when(e == 0)
    def _():
        for j in range(NBUF):
            issue(j, j)

    pltpu.make_async_copy(w0_hbm.at[e], w0_buf.at[slot], sems.at[0, slot]).wait()
    pltpu.make_async_copy(w1_hbm.at[e], w1_buf.at[slot], sems.at[1, slot]).wait()
    pltpu.make_async_copy(w2_hbm.at[e], w2_buf.at[slot], sems.at[2, slot]).wait()

    x = x_ref[...].astype(jnp.bfloat16)
    dn = (((1,), (1,)), ((), ()))

    def mm(a, w):
        return jax.lax.dot_general(a, w.astype(jnp.bfloat16), dn,
                                   preferred_element_type=jnp.float32)

    g = mm(x, w0_buf[slot])
    g = g * s0_ref[e]
    u = mm(x, w1_buf[slot])
    h = ((g * jax.nn.sigmoid(g)) * u).astype(jnp.bfloat16)
    y = mm(h, w2_buf[slot])

    se = se_ref[...]
    rw = rw_ref[...]
    cw = jnp.sum(jnp.where(se == e, rw, 0.0), axis=1, keepdims=True)
    contrib = y * (cw * (s1_ref[e] * s2_ref[e]))

    @pl.when(e == 0)
    def _():
        o_ref[...] = contrib

    @pl.when(e != 0)
    def _():
        o_ref[...] += contrib

    nxt = e + NBUF

    @pl.when(nxt < E)
    def _():
        issue(nxt, slot)


def kernel(x, w0, w1, w2, s0, s1, s2, selected_experts, routing_weights,
           gathered_experts_out_buf, select_experts_middle, routing_weights_middle,
           gather_buffer, scatter_buffer, use_ppl):
    se = selected_experts.astype(jnp.int32)
    out = pl.pallas_call(
        _moe_kernel,
        grid=(E,),
        in_specs=[
            pl.BlockSpec((T, K), lambda e: (0, 0)),
            pl.BlockSpec((T, K), lambda e: (0, 0)),
            pl.BlockSpec(memory_space=pltpu.SMEM),
            pl.BlockSpec(memory_space=pltpu.SMEM),
            pl.BlockSpec(memory_space=pltpu.SMEM),
            pl.BlockSpec((T, D), lambda e: (0, 0)),
            pl.BlockSpec(memory_space=pl.ANY),
            pl.BlockSpec(memory_space=pl.ANY),
            pl.BlockSpec(memory_space=pl.ANY),
        ],
        out_specs=pl.BlockSpec((T, D), lambda e: (0, 0)),
        out_shape=jax.ShapeDtypeStruct((T, D), jnp.float32),
        scratch_shapes=[
            pltpu.VMEM((NBUF, F, D), jnp.float32),
            pltpu.VMEM((NBUF, F, D), jnp.float32),
            pltpu.VMEM((NBUF, D, F), jnp.float32),
            pltpu.SemaphoreType.DMA((3, NBUF)),
        ],
    )(se, routing_weights, s0, s1, s2, x, w0, w1, w2)
    return out


# speedup vs baseline: 2.0503x; 2.0503x over previous
import jax
import jax.numpy as jnp
from jax.experimental import pallas as pl
from jax.experimental.pallas import tpu as pltpu

T, D, F, E, K = 32, 1024, 704, 64, 8


def _moe_kernel(se_ref, rw_ref, s0_ref, s1_ref, s2_ref, x_ref,
                w0_ref, w1_ref, w2t_ref, o_ref):
    e = pl.program_id(0)
    x = x_ref[...].astype(jnp.bfloat16)
    dn_t = (((1,), (1,)), ((), ()))
    dn_n = (((1,), (0,)), ((), ()))
    w0e = w0_ref[0].astype(jnp.bfloat16)
    w1e = w1_ref[0].astype(jnp.bfloat16)
    w2e = w2t_ref[0].astype(jnp.bfloat16)
    g = jax.lax.dot_general(x, w0e, dn_t, preferred_element_type=jnp.float32)
    g = g * s0_ref[e]
    u = jax.lax.dot_general(x, w1e, dn_t, preferred_element_type=jnp.float32)
    h = ((g * jax.nn.sigmoid(g)) * u).astype(jnp.bfloat16)
    y = jax.lax.dot_general(h, w2e, dn_n, preferred_element_type=jnp.float32)
    se = se_ref[...]
    rw = rw_ref[...]
    cw = jnp.sum(jnp.where(se == e, rw, 0.0), axis=1, keepdims=True)
    contrib = y * (cw * (s1_ref[e] * s2_ref[e]))

    @pl.when(e == 0)
    def _():
        o_ref[...] = contrib

    @pl.when(e != 0)
    def _():
        o_ref[...] += contrib


def kernel(x, w0, w1, w2, s0, s1, s2, selected_experts, routing_weights,
           gathered_experts_out_buf, select_experts_middle, routing_weights_middle,
           gather_buffer, scatter_buffer, use_ppl):
    se = selected_experts.astype(jnp.int32)
    w2t = jnp.swapaxes(w2, 1, 2)
    out = pl.pallas_call(
        _moe_kernel,
        grid=(E,),
        in_specs=[
            pl.BlockSpec((T, K), lambda e: (0, 0)),
            pl.BlockSpec((T, K), lambda e: (0, 0)),
            pl.BlockSpec(memory_space=pltpu.SMEM),
            pl.BlockSpec(memory_space=pltpu.SMEM),
            pl.BlockSpec(memory_space=pltpu.SMEM),
            pl.BlockSpec((T, D), lambda e: (0, 0)),
            pl.BlockSpec((1, F, D), lambda e: (e, 0, 0)),
            pl.BlockSpec((1, F, D), lambda e: (e, 0, 0)),
            pl.BlockSpec((1, F, D), lambda e: (e, 0, 0)),
        ],
        out_specs=pl.BlockSpec((T, D), lambda e: (0, 0)),
        out_shape=jax.ShapeDtypeStruct((T, D), jnp.float32),
    )(se, routing_weights, s0, s1, s2, x, w0, w1, w2t)
    return out
